# baseline (speedup 1.0000x reference)
import functools

import jax
import jax.experimental.layout
import jax.numpy as jnp
from jax import lax
from jax.experimental import pallas as pl
from jax.experimental.pallas import tpu as pltpu
from jax.experimental.pallas import tpu_sc as plsc

EMB = 64
EMBW = EMB // 2
LANES = 16
CHUNK = 128


def _scores_body(ent_hbm, rel_hbm, h_hbm, r_hbm, t_hbm, dummy_hbm, out_hbm,
                 idx_v, h_rows, r_rows, t_rows, out_v, sem,
                 *, n_per_worker):
    nc = 2
    wid = lax.axis_index("s") * nc + lax.axis_index("c")
    lane = lax.broadcasted_iota(jnp.int32, (LANES,), 0)
    dnums = lax.GatherDimensionNumbers(
        offset_dims=(), collapsed_slice_dims=(0,), start_index_map=(0,))

    def fold(x, d):
        shuf = lax.gather(x, (lane ^ d)[:, None], dnums, (1,),
                          mode=lax.GatherScatterMode.PROMISE_IN_BOUNDS)
        return x + shuf

    def chunk_body(chunk, _):
        base = wid * n_per_worker + chunk * CHUNK
        pltpu.sync_copy(h_hbm.at[pl.ds(base, CHUNK)], idx_v.at[0])
        pltpu.sync_copy(r_hbm.at[pl.ds(base, CHUNK)], idx_v.at[1])
        pltpu.sync_copy(t_hbm.at[pl.ds(base, CHUNK)], idx_v.at[2])

        def fire(g, _):
            hvec = idx_v[0, pl.ds(g * LANES, LANES)]
            rvec = idx_v[1, pl.ds(g * LANES, LANES)]
            tvec = idx_v[2, pl.ds(g * LANES, LANES)]
            for i in range(LANES):
                j = g * LANES + i
                ho = pl.multiple_of(hvec[i] * EMBW, EMBW)
                ro = pl.multiple_of(rvec[i] * EMBW, EMBW)
                to = pl.multiple_of(tvec[i] * EMBW, EMBW)
                pltpu.async_copy(ent_hbm.at[pl.ds(ho, EMBW)],
                                 h_rows.at[pl.ds(j * EMBW, EMBW)], sem)
                pltpu.async_copy(rel_hbm.at[pl.ds(ro, EMBW)],
                                 r_rows.at[pl.ds(j * EMBW, EMBW)], sem)
                pltpu.async_copy(ent_hbm.at[pl.ds(to, EMBW)],
                                 t_rows.at[pl.ds(j * EMBW, EMBW)], sem)
            return 0

        lax.fori_loop(0, CHUNK // LANES, fire, 0)
        pltpu.make_async_copy(dummy_hbm, h_rows, sem).wait()
        pltpu.make_async_copy(dummy_hbm, r_rows, sem).wait()
        pltpu.make_async_copy(dummy_hbm, t_rows, sem).wait()

        def grp(g, _):
            parts = []
            for i in range(LANES):
                off = (g * LANES + i) * EMBW
                p = None
                for k in range(EMBW // LANES):
                    hw = plsc.bitcast(
                        h_rows[pl.ds(off + k * LANES, LANES)], jnp.bfloat16)
                    rw = plsc.bitcast(
                        r_rows[pl.ds(off + k * LANES, LANES)], jnp.bfloat16)
                    tw = plsc.bitcast(
                        t_rows[pl.ds(off + k * LANES, LANES)], jnp.bfloat16)
                    ha, hb = plsc.unpack(
                        hw, format=plsc.PackFormat.INTERLEAVED)
                    ra, rb = plsc.unpack(
                        rw, format=plsc.PackFormat.INTERLEAVED)
                    ta, tb = plsc.unpack(
                        tw, format=plsc.PackFormat.INTERLEAVED)
                    prod = ha * ra * ta + hb * rb * tb
                    p = prod if p is None else p + prod
                parts.append(p)
            d = 1
            while len(parts) > 1:
                sel = (lane & d) == 0
                parts = [jnp.where(sel, fold(a, d), fold(b, d))
                         for a, b in zip(parts[0::2], parts[1::2])]
                d *= 2
            out_v[pl.ds(g * LANES, LANES)] = parts[0]
            return 0

        lax.fori_loop(0, CHUNK // LANES, grp, 0)
        pltpu.sync_copy(out_v, out_hbm.at[pl.ds(base, CHUNK)])
        return 0

    lax.fori_loop(0, n_per_worker // CHUNK, chunk_body, 0)


def _make_scores(total):
    info = plsc.get_sparse_core_info()
    nw = info.num_cores * info.num_subcores
    assert total % (nw * CHUNK) == 0
    n_per_worker = total // nw
    mesh = plsc.VectorSubcoreMesh(core_axis_name="c", subcore_axis_name="s")

    return pl.kernel(
        functools.partial(_scores_body, n_per_worker=n_per_worker),
        mesh=mesh,
        compiler_params=pltpu.CompilerParams(needs_layout_passes=False),
        out_type=jax.ShapeDtypeStruct((total,), jnp.float32),
        scratch_types=[
            pltpu.VMEM((3, CHUNK), jnp.int32),
            pltpu.VMEM((CHUNK * EMBW,), jnp.int32),
            pltpu.VMEM((CHUNK * EMBW,), jnp.int32),
            pltpu.VMEM((CHUNK * EMBW,), jnp.int32),
            pltpu.VMEM((CHUNK,), jnp.float32),
            pltpu.SemaphoreType.DMA,
        ],
    )


def kernel(entity_emb, relation_emb, pos_h, pos_r, pos_t, neg_h, neg_r, neg_t):
    batch = pos_h.shape[0]
    h = jnp.concatenate([pos_h, neg_h]).astype(jnp.int32)
    r = jnp.concatenate([pos_r, neg_r]).astype(jnp.int32)
    t = jnp.concatenate([pos_t, neg_t]).astype(jnp.int32)
    row_major = jax.experimental.layout.Layout(major_to_minor=(0, 1))
    ent = lax.bitcast_convert_type(
        entity_emb.astype(jnp.bfloat16).reshape(-1, EMBW, 2), jnp.int32)
    rel = lax.bitcast_convert_type(
        relation_emb.astype(jnp.bfloat16).reshape(-1, EMBW, 2), jnp.int32)
    ent = jax.experimental.layout.with_layout_constraint(ent, row_major)
    rel = jax.experimental.layout.with_layout_constraint(rel, row_major)
    dummy = jnp.zeros((CHUNK * EMBW,), jnp.int32)
    scores = _make_scores(2 * batch)(
        jnp.reshape(ent, (-1,)), jnp.reshape(rel, (-1,)), h, r, t, dummy)
    return scores[:batch], scores[batch:]


# speedup vs baseline: 4.3065x; 4.3065x over previous
import functools

import jax
import jax.numpy as jnp
from jax import lax
from jax.experimental import pallas as pl
from jax.experimental.pallas import tpu as pltpu
from jax.experimental.pallas import tpu_sc as plsc

EMB = 64
LANES = 16
CHUNK = 128


def _scores_body(ent_hbm, rel_hbm, h_hbm, r_hbm, t_hbm, out_hbm,
                 idx0, idx1, h0, r0, t0, h1, r1, t1, out_v, sem0, sem1,
                 *, n_per_worker):
    nc = 2
    wid = lax.axis_index("s") * nc + lax.axis_index("c")
    lane = lax.broadcasted_iota(jnp.int32, (LANES,), 0)
    dnums = lax.GatherDimensionNumbers(
        offset_dims=(), collapsed_slice_dims=(0,), start_index_map=(0,))
    bufs = ((idx0, h0, r0, t0, sem0), (idx1, h1, r1, t1, sem1))
    n_chunks = n_per_worker // CHUNK

    def fold(x, d):
        shuf = lax.gather(x, (lane ^ d)[:, None], dnums, (1,),
                          mode=lax.GatherScatterMode.PROMISE_IN_BOUNDS)
        return x + shuf

    def stage_and_fire(chunk, slot):
        idx_v, h_rows, r_rows, t_rows, sem = bufs[slot]
        base = wid * n_per_worker + chunk * CHUNK
        pltpu.sync_copy(h_hbm.at[pl.ds(base, CHUNK)],
                        idx_v.at[pl.ds(0, CHUNK)])
        pltpu.sync_copy(r_hbm.at[pl.ds(base, CHUNK)],
                        idx_v.at[pl.ds(CHUNK, CHUNK)])
        pltpu.sync_copy(t_hbm.at[pl.ds(base, CHUNK)],
                        idx_v.at[pl.ds(2 * CHUNK, CHUNK)])

        def fire(g, _):
            hvec = idx_v[pl.ds(g * LANES, LANES)]
            rvec = idx_v[pl.ds(CHUNK + g * LANES, LANES)]
            tvec = idx_v[pl.ds(2 * CHUNK + g * LANES, LANES)]
            for i in range(LANES):
                j = g * LANES + i
                pltpu.async_copy(ent_hbm.at[hvec[i]], h_rows.at[j], sem)
                pltpu.async_copy(rel_hbm.at[rvec[i]], r_rows.at[j], sem)
                pltpu.async_copy(ent_hbm.at[tvec[i]], t_rows.at[j], sem)
            return 0

        lax.fori_loop(0, CHUNK // LANES, fire, 0)

    def drain(slot):
        _, h_rows, r_rows, t_rows, sem = bufs[slot]
        pltpu.make_async_copy(ent_hbm.at[pl.ds(0, CHUNK)], h_rows, sem).wait()
        pltpu.make_async_copy(ent_hbm.at[pl.ds(0, CHUNK)], r_rows, sem).wait()
        pltpu.make_async_copy(ent_hbm.at[pl.ds(0, CHUNK)], t_rows, sem).wait()

    def compute(chunk, slot):
        _, h_rows, r_rows, t_rows, _ = bufs[slot]
        base = wid * n_per_worker + chunk * CHUNK

        def grp(g, _):
            parts = []
            for i in range(LANES):
                idx = g * LANES + i
                p = (h_rows[idx, pl.ds(0, LANES)]
                     * r_rows[idx, pl.ds(0, LANES)]
                     * t_rows[idx, pl.ds(0, LANES)])
                for k in range(1, EMB // LANES):
                    p = p + (h_rows[idx, pl.ds(k * LANES, LANES)]
                             * r_rows[idx, pl.ds(k * LANES, LANES)]
                             * t_rows[idx, pl.ds(k * LANES, LANES)])
                parts.append(p)
            d = 1
            while len(parts) > 1:
                sel = (lane & d) == 0
                parts = [jnp.where(sel, fold(a, d), fold(b, d))
                         for a, b in zip(parts[0::2], parts[1::2])]
                d *= 2
            out_v[pl.ds(g * LANES, LANES)] = parts[0]
            return 0

        lax.fori_loop(0, CHUNK // LANES, grp, 0)
        pltpu.sync_copy(out_v, out_hbm.at[pl.ds(base, CHUNK)])

    assert n_chunks % 2 == 0
    stage_and_fire(0, 0)

    def chunk_pair(c2, _):
        c0 = c2 * 2
        drain(0)
        stage_and_fire(c0 + 1, 1)
        compute(c0, 0)
        drain(1)

        @pl.when(c2 + 1 < n_chunks // 2)
        def _():
            stage_and_fire(c0 + 2, 0)

        compute(c0 + 1, 1)
        return 0

    lax.fori_loop(0, n_chunks // 2, chunk_pair, 0)


def _make_scores(total):
    info = plsc.get_sparse_core_info()
    nw = info.num_cores * info.num_subcores
    assert total % (nw * CHUNK) == 0
    n_per_worker = total // nw
    mesh = plsc.VectorSubcoreMesh(core_axis_name="c", subcore_axis_name="s")

    return pl.kernel(
        functools.partial(_scores_body, n_per_worker=n_per_worker),
        mesh=mesh,
        out_type=jax.ShapeDtypeStruct((total,), jnp.float32),
        scratch_types=[
            pltpu.VMEM((3 * CHUNK,), jnp.int32),
            pltpu.VMEM((3 * CHUNK,), jnp.int32),
            pltpu.VMEM((CHUNK, EMB), jnp.float32),
            pltpu.VMEM((CHUNK, EMB), jnp.float32),
            pltpu.VMEM((CHUNK, EMB), jnp.float32),
            pltpu.VMEM((CHUNK, EMB), jnp.float32),
            pltpu.VMEM((CHUNK, EMB), jnp.float32),
            pltpu.VMEM((CHUNK, EMB), jnp.float32),
            pltpu.VMEM((CHUNK,), jnp.float32),
            pltpu.SemaphoreType.DMA,
            pltpu.SemaphoreType.DMA,
        ],
    )


def kernel(entity_emb, relation_emb, pos_h, pos_r, pos_t, neg_h, neg_r, neg_t):
    batch = pos_h.shape[0]
    h = jnp.concatenate([pos_h, neg_h]).astype(jnp.int32)
    r = jnp.concatenate([pos_r, neg_r]).astype(jnp.int32)
    t = jnp.concatenate([pos_t, neg_t]).astype(jnp.int32)
    scores = _make_scores(2 * batch)(entity_emb, relation_emb, h, r, t)
    return scores[:batch], scores[batch:]
